# baseline (speedup 1.0000x reference)
import functools

import numpy as np
import jax
import jax.numpy as jnp
from jax import lax
from jax.experimental import pallas as pl
from jax.experimental.pallas import tpu as pltpu
from jax.experimental.pallas import tpu_sc as plsc

B = 4096
NF = 26
VOC = 100001
E = 32
NV = NF + 1
NI = NV * (NV - 1) // 2

_NC, _NS = 2, 16
_NW = _NC * _NS
_ROWS = B * NF
_CHUNK = 128
_NCHUNK = _ROWS // _CHUNK
_CPW = _NCHUNK // _NW


def _sc_gather(table_flat, idx3d):
    mesh = plsc.VectorSubcoreMesh(core_axis_name="c", subcore_axis_name="s")

    @functools.partial(
        pl.kernel,
        mesh=mesh,
        out_type=jax.ShapeDtypeStruct((_NCHUNK, _CHUNK, E), jnp.float32),
        scratch_types=[
            pltpu.VMEM((_CPW, _CHUNK), jnp.int32),
            pltpu.VMEM((_CPW, _CHUNK, E), jnp.float32),
            pltpu.SemaphoreType.DMA,
        ],
        compiler_params=pltpu.CompilerParams(use_tc_tiling_on_sc=False),
    )
    def k(table_hbm, idx_hbm, out_hbm, idx_v, rows_v, sem):
        wid = lax.axis_index("s") * _NC + lax.axis_index("c")
        base = wid * _CPW
        pltpu.sync_copy(idx_hbm.at[wid], idx_v)
        for half in range(2):
            copies = [
                pltpu.async_copy(
                    table_hbm.at[idx_v.at[half * 13 + j]],
                    rows_v.at[half * 13 + j],
                    sem,
                )
                for j in range(13)
            ]
            for c in copies:
                c.wait()
        pltpu.sync_copy(rows_v, out_hbm.at[pl.ds(base, _CPW)])

    return k(table_flat, idx3d)


def _bottom_body(x_ref, w0, b0, g0, be0, w1, b1, g1, be1, w2, b2, g2, be2,
                 out_ref):
    h = x_ref[...]
    for w, b, g, be in ((w0, b0, g0, be0), (w1, b1, g1, be1),
                        (w2, b2, g2, be2)):
        h = jnp.dot(h, w[...], preferred_element_type=jnp.float32) + b[...]
        m = jnp.mean(h, axis=0, keepdims=True)
        c = h - m
        v = jnp.mean(c * c, axis=0, keepdims=True)
        h = g[...] * c * lax.rsqrt(v + 1e-5) + be[...]
        h = jnp.maximum(h, 0.0)
    out_ref[...] = h


def _bottom_mlp(x, p):
    args = [x]
    for i in range(3):
        args += [p[f"bw{i}"].astype(jnp.float32),
                 p[f"bb{i}"].reshape(1, -1), p[f"bg{i}"].reshape(1, -1),
                 p[f"bbeta{i}"].reshape(1, -1)]
    return pl.pallas_call(
        _bottom_body,
        out_shape=jax.ShapeDtypeStruct((B, E), jnp.float32),
    )(*args)


_BB = 512


def _top_body(h2_ref, embs_ref, w32t, m729t, tb0, tw1t, tb1, tw2t, tb2,
              tw3t, tb3, out_ref):
    h2t = h2_ref[...].T
    at = jnp.concatenate([h2t, embs_ref[...].T], axis=0)
    a3 = at.reshape(NV, E, _BB)
    rows = []
    for n in range(NV):
        prod = a3 * a3[n][None]
        rows.append(jnp.sum(prod, axis=1))
    gt = jnp.concatenate(rows, axis=0)
    t = jnp.dot(w32t[...], h2t, preferred_element_type=jnp.float32)
    t = t + jnp.dot(m729t[...], gt, preferred_element_type=jnp.float32)
    t = jnp.maximum(t + tb0[...], 0.0)
    t = jnp.maximum(jnp.dot(tw1t[...], t, preferred_element_type=jnp.float32)
                    + tb1[...], 0.0)
    t = jnp.maximum(jnp.dot(tw2t[...], t, preferred_element_type=jnp.float32)
                    + tb2[...], 0.0)
    o = jnp.dot(tw3t[...], t, preferred_element_type=jnp.float32) + tb3[...]
    out_ref[...] = o[0, :]


def _top_mlp(h2, embs, p):
    rows, cols = np.triu_indices(NV, k=1)
    tw0 = p["tw0"].astype(jnp.float32)
    w32t = tw0[:E, :].T
    m729 = jnp.zeros((NV * NV, tw0.shape[1]), jnp.float32)
    m729t = m729.at[rows * NV + cols, :].set(tw0[E:, :]).T
    weights = (w32t, m729t, p["tb0"].reshape(-1, 1),
               p["tw1"].T, p["tb1"].reshape(-1, 1),
               p["tw2"].T, p["tb2"].reshape(-1, 1),
               p["tw3"].T, p["tb3"].reshape(-1, 1))
    wspec = [pl.BlockSpec(w.shape, lambda i: (0, 0)) for w in weights]
    return pl.pallas_call(
        _top_body,
        grid=(B // _BB,),
        in_specs=[pl.BlockSpec((_BB, E), lambda i: (i, 0)),
                  pl.BlockSpec((_BB, NF * E), lambda i: (i, 0))] + wspec,
        out_specs=pl.BlockSpec((_BB,), lambda i: (i,)),
        out_shape=jax.ShapeDtypeStruct((B,), jnp.float32),
    )(h2, embs, *weights)


def kernel(dense_inputs, sparse_inputs, params):
    table_flat = params["tables"].astype(jnp.float32).reshape(NF * VOC, E)
    idx = (sparse_inputs.astype(jnp.int32)
           + (jnp.arange(NF, dtype=jnp.int32) * VOC)[None, :])
    idx3d = idx.reshape(_NW, _CPW, _CHUNK)
    embs = _sc_gather(table_flat, idx3d).reshape(B, NF * E)
    h2 = _bottom_mlp(dense_inputs.astype(jnp.float32), params)
    return _top_mlp(h2, embs, params)


# speedup vs baseline: 2.4723x; 2.4723x over previous
import functools

import numpy as np
import jax
import jax.numpy as jnp
from jax import lax
from jax.experimental import pallas as pl
from jax.experimental.pallas import tpu as pltpu
from jax.experimental.pallas import tpu_sc as plsc

B = 4096
NF = 26
VOC = 100001
E = 32
NV = NF + 1
NI = NV * (NV - 1) // 2

_NC, _NS = 2, 16
_NW = _NC * _NS
_ROWS = B * NF
_CHUNK = 128
_NCHUNK = _ROWS // _CHUNK
_CPW = _NCHUNK // _NW


def _sc_gather(tables, idxt):
    mesh = plsc.VectorSubcoreMesh(core_axis_name="c", subcore_axis_name="s")

    @functools.partial(
        pl.kernel,
        mesh=mesh,
        out_type=jax.ShapeDtypeStruct((_NCHUNK, _CHUNK, E), jnp.float32),
        scratch_types=[
            pltpu.VMEM((_CPW, _CHUNK), jnp.int32),
            pltpu.VMEM((_CPW, _CHUNK, E), jnp.float32),
            pltpu.SemaphoreType.DMA,
            pltpu.SemaphoreType.DMA,
        ],
        compiler_params=pltpu.CompilerParams(use_tc_tiling_on_sc=False),
    )
    def k(table_hbm, idx_hbm, out_hbm, idx_v, rows_v, isem, gsem):
        wid = lax.axis_index("s") * _NC + lax.axis_index("c")
        base = wid * _CPW
        fields = [(base + j) // 32 for j in range(_CPW)]
        blks = [(base + j) % 32 for j in range(_CPW)]
        icopies = [
            pltpu.async_copy(idx_hbm.at[fields[j], blks[j]], idx_v.at[j], isem)
            for j in range(_CPW)
        ]
        for j in range(_CPW):
            icopies[j].wait()
        for half in range(2):
            copies = [
                pltpu.async_copy(
                    table_hbm.at[fields[half * 13 + j]].at[
                        idx_v.at[half * 13 + j]],
                    rows_v.at[half * 13 + j],
                    gsem,
                )
                for j in range(13)
            ]
            for c in copies:
                c.wait()
        pltpu.sync_copy(rows_v, out_hbm.at[pl.ds(base, _CPW)])

    return k(tables, idxt)


def _bottom_body(x_ref, w0, b0, g0, be0, w1, b1, g1, be1, w2, b2, g2, be2,
                 out_ref):
    h = x_ref[...]
    for w, b, g, be in ((w0, b0, g0, be0), (w1, b1, g1, be1),
                        (w2, b2, g2, be2)):
        h = jnp.dot(h, w[...], preferred_element_type=jnp.float32) + b[...]
        m = jnp.mean(h, axis=0, keepdims=True)
        c = h - m
        v = jnp.mean(c * c, axis=0, keepdims=True)
        h = g[...] * c * lax.rsqrt(v + 1e-5) + be[...]
        h = jnp.maximum(h, 0.0)
    out_ref[...] = h


def _bottom_mlp(x, p):
    args = [x]
    for i in range(3):
        args += [p[f"bw{i}"].astype(jnp.float32),
                 p[f"bb{i}"].reshape(1, -1), p[f"bg{i}"].reshape(1, -1),
                 p[f"bbeta{i}"].reshape(1, -1)]
    return pl.pallas_call(
        _bottom_body,
        out_shape=jax.ShapeDtypeStruct((B, E), jnp.float32),
    )(*args)


_BB = 512


def _top_body(h2_ref, embs_ref, w32t, m729t, tb0, tw1t, tb1, tw2t, tb2,
              tw3t, tb3, out_ref):
    h2t = h2_ref[...].T
    at = jnp.concatenate([h2t, embs_ref[...].T], axis=0)
    a3 = at.reshape(NV, E, _BB)
    rows = []
    for n in range(NV):
        prod = a3 * a3[n][None]
        rows.append(jnp.sum(prod, axis=1))
    gt = jnp.concatenate(rows, axis=0)
    t = jnp.dot(w32t[...], h2t, preferred_element_type=jnp.float32)
    t = t + jnp.dot(m729t[...], gt, preferred_element_type=jnp.float32)
    t = jnp.maximum(t + tb0[...], 0.0)
    t = jnp.maximum(jnp.dot(tw1t[...], t, preferred_element_type=jnp.float32)
                    + tb1[...], 0.0)
    t = jnp.maximum(jnp.dot(tw2t[...], t, preferred_element_type=jnp.float32)
                    + tb2[...], 0.0)
    o = jnp.dot(tw3t[...], t, preferred_element_type=jnp.float32) + tb3[...]
    out_ref[...] = o[0, :]


def _top_mlp(h2, embs, p):
    rows, cols = np.triu_indices(NV, k=1)
    tw0 = p["tw0"].astype(jnp.float32)
    w32t = tw0[:E, :].T
    m729 = jnp.zeros((NV * NV, tw0.shape[1]), jnp.float32)
    m729t = m729.at[rows * NV + cols, :].set(tw0[E:, :]).T
    weights = (w32t, m729t, p["tb0"].reshape(-1, 1),
               p["tw1"].T, p["tb1"].reshape(-1, 1),
               p["tw2"].T, p["tb2"].reshape(-1, 1),
               p["tw3"].T, p["tb3"].reshape(-1, 1))
    wspec = [pl.BlockSpec(w.shape, lambda i: (0, 0)) for w in weights]
    return pl.pallas_call(
        _top_body,
        grid=(B // _BB,),
        in_specs=[pl.BlockSpec((_BB, E), lambda i: (i, 0)),
                  pl.BlockSpec((_BB, NF * E), lambda i: (i, 0))] + wspec,
        out_specs=pl.BlockSpec((_BB,), lambda i: (i,)),
        out_shape=jax.ShapeDtypeStruct((B,), jnp.float32),
    )(h2, embs, *weights)


def kernel(dense_inputs, sparse_inputs, params):
    idxt = sparse_inputs.astype(jnp.int32).T.reshape(NF, 32, _CHUNK)
    out = _sc_gather(params["tables"], idxt)
    embs = (out.reshape(NF, 32, _CHUNK, E)
            .transpose(1, 2, 0, 3).reshape(B, NF * E))
    h2 = _bottom_mlp(dense_inputs, params)
    return _top_mlp(h2, embs, params)


# speedup vs baseline: 13.5466x; 5.4793x over previous
import functools

import numpy as np
import jax
import jax.numpy as jnp
from jax import lax
from jax.experimental import pallas as pl
from jax.experimental.pallas import tpu as pltpu
from jax.experimental.pallas import tpu_sc as plsc

B = 4096
NF = 26
VOC = 100001
E = 32
NV = NF + 1
NI = NV * (NV - 1) // 2

QS = 3200
VCHUNK = 4 * QS
NCH = 8
SRF = NCH * QS

_NC, _NS = 2, 16
_NW = _NC * _NS
_CHUNK = 128
_NCHUNK = B * NF // _CHUNK
_CPW = _NCHUNK // _NW


def _repack_body(tt_ref, out_ref):
    x = tt_ref[0]
    xt = x.T
    out_ref[0] = jnp.concatenate(
        [xt[0:QS], xt[QS:2 * QS], xt[2 * QS:3 * QS], xt[3 * QS:4 * QS]],
        axis=1)


def _repack(tables):
    tt = jnp.transpose(tables, (0, 2, 1))
    return pl.pallas_call(
        _repack_body,
        grid=(NF, NCH),
        in_specs=[pl.BlockSpec((1, E, VCHUNK), lambda f, c: (f, 0, c))],
        out_specs=pl.BlockSpec((1, QS, 128), lambda f, c: (f, c, 0)),
        out_shape=jax.ShapeDtypeStruct((NF, SRF, 128), jnp.float32),
    )(tt)


def _sc_gather(pack, srt):
    mesh = plsc.VectorSubcoreMesh(core_axis_name="c", subcore_axis_name="s")

    @functools.partial(
        pl.kernel,
        mesh=mesh,
        out_type=jax.ShapeDtypeStruct((NF, B, 128), jnp.float32),
        scratch_types=[
            pltpu.VMEM((_CHUNK,), jnp.int32),
            pltpu.VMEM((_CHUNK, 128), jnp.float32),
            pltpu.SemaphoreType.DMA,
        ],
        compiler_params=pltpu.CompilerParams(use_tc_tiling_on_sc=False),
    )
    def k(pack_hbm, srt_hbm, out_hbm, idx_v, buf_v, gsem):
        wid = lax.axis_index("s") * _NC + lax.axis_index("c")
        base = wid * _CPW

        def body(j, carry):
            c = base + j
            f = c // 32
            blk = c % 32
            pltpu.sync_copy(srt_hbm.at[f, blk], idx_v)
            pltpu.async_copy(pack_hbm.at[f].at[idx_v], buf_v, gsem).wait()
            pltpu.sync_copy(buf_v, out_hbm.at[f, pl.ds(blk * _CHUNK, _CHUNK)])
            return carry

        lax.fori_loop(0, _CPW, body, 0)

    return k(pack, srt)


def _bottom_body(x_ref, w0, b0, g0, be0, w1, b1, g1, be1, w2, b2, g2, be2,
                 out_ref):
    h = x_ref[...]
    for w, b, g, be in ((w0, b0, g0, be0), (w1, b1, g1, be1),
                        (w2, b2, g2, be2)):
        h = jnp.dot(h, w[...], preferred_element_type=jnp.float32) + b[...]
        m = jnp.mean(h, axis=0, keepdims=True)
        c = h - m
        v = jnp.mean(c * c, axis=0, keepdims=True)
        h = g[...] * c * lax.rsqrt(v + 1e-5) + be[...]
        h = jnp.maximum(h, 0.0)
    out_ref[...] = h


def _bottom_mlp(x, p):
    args = [x]
    for i in range(3):
        args += [p[f"bw{i}"], p[f"bb{i}"].reshape(1, -1),
                 p[f"bg{i}"].reshape(1, -1), p[f"bbeta{i}"].reshape(1, -1)]
    return pl.pallas_call(
        _bottom_body,
        out_shape=jax.ShapeDtypeStruct((B, E), jnp.float32),
    )(*args)


_BB = 256


def _top_body(h2_ref, embs_ref, offs_ref, w32t, m729t, tb0, tw1t, tb1, tw2t,
              tb2, tw3t, tb3, out_ref):
    h2t = h2_ref[...].T
    offs = offs_ref[...]
    parts = [h2t]
    for f in range(NF):
        e = embs_ref[f]
        o = offs[:, f:f + 1]
        x32 = jnp.where(o == 0, e[:, 0:E], 0.0)
        x32 = x32 + jnp.where(o == 1, e[:, E:2 * E], 0.0)
        x32 = x32 + jnp.where(o == 2, e[:, 2 * E:3 * E], 0.0)
        x32 = x32 + jnp.where(o == 3, e[:, 3 * E:4 * E], 0.0)
        parts.append(x32.T)
    at = jnp.concatenate(parts, axis=0)
    a3 = at.reshape(NV, E, _BB)
    rows = []
    for n in range(NV):
        prod = a3 * a3[n][None]
        rows.append(jnp.sum(prod, axis=1))
    gt = jnp.concatenate(rows, axis=0)
    t = jnp.dot(w32t[...], h2t, preferred_element_type=jnp.float32)
    t = t + jnp.dot(m729t[...], gt, preferred_element_type=jnp.float32)
    t = jnp.maximum(t + tb0[...], 0.0)
    t = jnp.maximum(jnp.dot(tw1t[...], t, preferred_element_type=jnp.float32)
                    + tb1[...], 0.0)
    t = jnp.maximum(jnp.dot(tw2t[...], t, preferred_element_type=jnp.float32)
                    + tb2[...], 0.0)
    o = jnp.dot(tw3t[...], t, preferred_element_type=jnp.float32) + tb3[...]
    out_ref[...] = o[0, :]


def _top_mlp(h2, embs128, offs, p):
    rows, cols = np.triu_indices(NV, k=1)
    tw0 = p["tw0"]
    w32t = tw0[:E, :].T
    m729 = jnp.zeros((NV * NV, tw0.shape[1]), jnp.float32)
    m729t = m729.at[rows * NV + cols, :].set(tw0[E:, :]).T
    weights = (w32t, m729t, p["tb0"].reshape(-1, 1),
               p["tw1"].T, p["tb1"].reshape(-1, 1),
               p["tw2"].T, p["tb2"].reshape(-1, 1),
               p["tw3"].T, p["tb3"].reshape(-1, 1))
    wspec = [pl.BlockSpec(w.shape, lambda i: (0, 0)) for w in weights]
    return pl.pallas_call(
        _top_body,
        grid=(B // _BB,),
        in_specs=[pl.BlockSpec((_BB, E), lambda i: (i, 0)),
                  pl.BlockSpec((NF, _BB, 128), lambda i: (0, i, 0)),
                  pl.BlockSpec((_BB, NF), lambda i: (i, 0))] + wspec,
        out_specs=pl.BlockSpec((_BB,), lambda i: (i,)),
        out_shape=jax.ShapeDtypeStruct((B,), jnp.float32),
    )(h2, embs128, offs, *weights)


def kernel(dense_inputs, sparse_inputs, params):
    vt = sparse_inputs.astype(jnp.int32).T
    c = vt // VCHUNK
    r = vt - c * VCHUNK
    srt = (c * QS + r % QS).reshape(NF, 32, _CHUNK)
    offs = (r // QS).T
    pack = _repack(params["tables"])
    embs128 = _sc_gather(pack, srt)
    h2 = _bottom_mlp(dense_inputs, params)
    return _top_mlp(h2, embs128, offs, params)


# speedup vs baseline: 14.3243x; 1.0574x over previous
import functools

import numpy as np
import jax
import jax.numpy as jnp
from jax import lax
from jax.experimental import pallas as pl
from jax.experimental.pallas import tpu as pltpu
from jax.experimental.pallas import tpu_sc as plsc

B = 4096
NF = 26
VOC = 100001
E = 32
NV = NF + 1
NI = NV * (NV - 1) // 2

QS = 3200
VCHUNK = 4 * QS
NCH = 8
SRF = NCH * QS

_NC, _NS = 2, 16
_NW = _NC * _NS
_CHUNK = 128
_NCHUNK = B * NF // _CHUNK
_CPW = _NCHUNK // _NW


def _repack_body(tt_ref, out_ref):
    x = tt_ref[0]
    xt = x.T
    out_ref[0] = jnp.concatenate(
        [xt[0:QS], xt[QS:2 * QS], xt[2 * QS:3 * QS], xt[3 * QS:4 * QS]],
        axis=1)


def _repack(tables):
    tt = jnp.transpose(tables, (0, 2, 1))
    return pl.pallas_call(
        _repack_body,
        grid=(NF, NCH),
        in_specs=[pl.BlockSpec((1, E, VCHUNK), lambda f, c: (f, 0, c))],
        out_specs=pl.BlockSpec((1, QS, 128), lambda f, c: (f, c, 0)),
        out_shape=jax.ShapeDtypeStruct((NF, SRF, 128), jnp.float32),
    )(tt)


def _sc_gather(pack, srt, offt):
    mesh = plsc.VectorSubcoreMesh(core_axis_name="c", subcore_axis_name="s")

    @functools.partial(
        pl.kernel,
        mesh=mesh,
        out_type=jax.ShapeDtypeStruct((NF, 32, E, 128), jnp.float32),
        scratch_types=[
            pltpu.VMEM((_CHUNK,), jnp.int32),
            pltpu.VMEM((_CHUNK,), jnp.int32),
            pltpu.VMEM((_CHUNK, 128), jnp.float32),
            pltpu.VMEM((E, 128), jnp.float32),
            pltpu.SemaphoreType.DMA,
        ],
        compiler_params=pltpu.CompilerParams(use_tc_tiling_on_sc=False,
                                             needs_layout_passes=False),
    )
    def k(pack_hbm, srt_hbm, offt_hbm, out_hbm, idx_v, off_v, buf_v, ebuf_v,
          gsem):
        wid = lax.axis_index("s") * _NC + lax.axis_index("c")
        base = wid * _CPW

        def body(j, carry):
            c = base + j
            f = c // 32
            blk = c % 32
            pltpu.sync_copy(srt_hbm.at[f, blk], idx_v)
            pltpu.sync_copy(offt_hbm.at[f, blk], off_v)
            pltpu.async_copy(pack_hbm.at[f].at[idx_v], buf_v, gsem).wait()

            for d in range(E):
                for g in range(8):
                    rows = jax.lax.iota(jnp.int32, 16) + g * 16
                    cols = off_v[pl.ds(g * 16, 16)] + d
                    vals = plsc.load_gather(buf_v, [rows, cols])
                    ebuf_v[d, pl.ds(g * 16, 16)] = vals
            pltpu.sync_copy(ebuf_v, out_hbm.at[f, blk])
            return carry

        lax.fori_loop(0, _CPW, body, 0)

    return k(pack, srt, offt)


def _bottom_body(x_ref, w0, b0, g0, be0, w1, b1, g1, be1, w2, b2, g2, be2,
                 out_ref):
    h = x_ref[...]
    for w, b, g, be in ((w0, b0, g0, be0), (w1, b1, g1, be1),
                        (w2, b2, g2, be2)):
        h = jnp.dot(h, w[...], preferred_element_type=jnp.float32) + b[...]
        m = jnp.mean(h, axis=0, keepdims=True)
        c = h - m
        v = jnp.mean(c * c, axis=0, keepdims=True)
        h = g[...] * c * lax.rsqrt(v + 1e-5) + be[...]
        h = jnp.maximum(h, 0.0)
    out_ref[...] = h


def _bottom_mlp(x, p):
    args = [x]
    for i in range(3):
        args += [p[f"bw{i}"], p[f"bb{i}"].reshape(1, -1),
                 p[f"bg{i}"].reshape(1, -1), p[f"bbeta{i}"].reshape(1, -1)]
    return pl.pallas_call(
        _bottom_body,
        out_shape=jax.ShapeDtypeStruct((B, E), jnp.float32),
    )(*args)


_BB = 128


def _top_body(h2_ref, embs_ref, w32t, m729t, tb0, tw1t, tb1, tw2t,
              tb2, tw3t, tb3, out_ref):
    h2t = h2_ref[...].T
    parts = [h2t] + [embs_ref[f, 0] for f in range(NF)]
    at = jnp.concatenate(parts, axis=0)
    a3 = at.reshape(NV, E, _BB)
    rows = []
    for n in range(NV):
        prod = a3 * a3[n][None]
        rows.append(jnp.sum(prod, axis=1))
    gt = jnp.concatenate(rows, axis=0)
    t = jnp.dot(w32t[...], h2t, preferred_element_type=jnp.float32)
    t = t + jnp.dot(m729t[...], gt, preferred_element_type=jnp.float32)
    t = jnp.maximum(t + tb0[...], 0.0)
    t = jnp.maximum(jnp.dot(tw1t[...], t, preferred_element_type=jnp.float32)
                    + tb1[...], 0.0)
    t = jnp.maximum(jnp.dot(tw2t[...], t, preferred_element_type=jnp.float32)
                    + tb2[...], 0.0)
    o = jnp.dot(tw3t[...], t, preferred_element_type=jnp.float32) + tb3[...]
    out_ref[...] = o[0, :]


def _top_mlp(h2, embs4, p):
    rows, cols = np.triu_indices(NV, k=1)
    tw0 = p["tw0"]
    w32t = tw0[:E, :].T
    m729 = jnp.zeros((NV * NV, tw0.shape[1]), jnp.float32)
    m729t = m729.at[rows * NV + cols, :].set(tw0[E:, :]).T
    weights = (w32t, m729t, p["tb0"].reshape(-1, 1),
               p["tw1"].T, p["tb1"].reshape(-1, 1),
               p["tw2"].T, p["tb2"].reshape(-1, 1),
               p["tw3"].T, p["tb3"].reshape(-1, 1))
    wspec = [pl.BlockSpec(w.shape, lambda i: (0, 0)) for w in weights]
    return pl.pallas_call(
        _top_body,
        grid=(B // _BB,),
        in_specs=[pl.BlockSpec((_BB, E), lambda i: (i, 0)),
                  pl.BlockSpec((NF, 1, E, 128), lambda i: (0, i, 0, 0))]
        + wspec,
        out_specs=pl.BlockSpec((_BB,), lambda i: (i,)),
        out_shape=jax.ShapeDtypeStruct((B,), jnp.float32),
    )(h2, embs4, *weights)


def kernel(dense_inputs, sparse_inputs, params):
    vt = sparse_inputs.astype(jnp.int32).T
    c = vt // VCHUNK
    r = vt - c * VCHUNK
    srt = (c * QS + r % QS).reshape(NF, 32, _CHUNK)
    offt = ((r // QS) * E).reshape(NF, 32, _CHUNK)
    pack = _repack(params["tables"])
    embs4 = _sc_gather(pack, srt, offt)
    h2 = _bottom_mlp(dense_inputs, params)
    return _top_mlp(h2, embs4, params)


# speedup vs baseline: 23.7158x; 1.6556x over previous
import functools

import numpy as np
import jax
import jax.numpy as jnp
from jax import lax
from jax.experimental import pallas as pl
from jax.experimental.pallas import tpu as pltpu
from jax.experimental.pallas import tpu_sc as plsc

B = 4096
NF = 26
VOC = 100001
E = 32
NV = NF + 1
NI = NV * (NV - 1) // 2

QS = 3200
VCHUNK = 4 * QS
NCH = 8
SRF = NCH * QS

_NC, _NS = 2, 16
_NW = _NC * _NS
_CHUNK = 128
_NCHUNK = B * NF // _CHUNK
_CPW = _NCHUNK // _NW


def _repack_body(b0, b1, b2, b3, out_ref):
    y = jnp.concatenate([b0[0], b1[0], b2[0], b3[0]], axis=0)
    out_ref[0] = y.T


def _repack(tables):
    tt = jnp.transpose(tables, (0, 2, 1))
    specs = [
        pl.BlockSpec((1, E, QS),
                     functools.partial(lambda q, f, c: (f, 0, 4 * c + q), q))
        for q in range(4)
    ]
    return pl.pallas_call(
        _repack_body,
        grid=(NF, NCH),
        in_specs=specs,
        out_specs=pl.BlockSpec((1, QS, 128), lambda f, c: (f, c, 0)),
        out_shape=jax.ShapeDtypeStruct((NF, SRF, 128), jnp.float32),
    )(tt, tt, tt, tt)


def _sc_gather(pack, srt, offt):
    mesh = plsc.VectorSubcoreMesh(core_axis_name="c", subcore_axis_name="s")

    @functools.partial(
        pl.kernel,
        mesh=mesh,
        out_type=jax.ShapeDtypeStruct((NF, 32, E, 128), jnp.float32),
        scratch_types=[
            pltpu.VMEM((_CHUNK,), jnp.int32),
            pltpu.VMEM((_CHUNK,), jnp.int32),
            pltpu.VMEM((_CHUNK, 128), jnp.float32),
            pltpu.VMEM((E, 128), jnp.float32),
            pltpu.SemaphoreType.DMA,
        ],
        compiler_params=pltpu.CompilerParams(use_tc_tiling_on_sc=False,
                                             needs_layout_passes=False),
    )
    def k(pack_hbm, srt_hbm, offt_hbm, out_hbm, idx_v, off_v, buf_v, ebuf_v,
          gsem):
        wid = lax.axis_index("s") * _NC + lax.axis_index("c")
        base = wid * _CPW

        def body(j, carry):
            c = base + j
            f = c // 32
            blk = c % 32
            pltpu.sync_copy(srt_hbm.at[f, blk], idx_v)
            pltpu.sync_copy(offt_hbm.at[f, blk], off_v)
            pltpu.async_copy(pack_hbm.at[f].at[idx_v], buf_v, gsem).wait()

            iota16 = jax.lax.iota(jnp.int32, 16)
            for g in range(8):
                rows = iota16 + g * 16
                off_g = off_v[pl.ds(g * 16, 16)]
                for d in range(E):
                    vals = plsc.load_gather(buf_v, [rows, off_g + d])
                    ebuf_v[d, pl.ds(g * 16, 16)] = vals
            pltpu.sync_copy(ebuf_v, out_hbm.at[f, blk])
            return carry

        lax.fori_loop(0, _CPW, body, 0)

    return k(pack, srt, offt)


def _bottom_body(x_ref, w0, b0, g0, be0, w1, b1, g1, be1, w2, b2, g2, be2,
                 out_ref):
    h = x_ref[...]
    for w, b, g, be in ((w0, b0, g0, be0), (w1, b1, g1, be1),
                        (w2, b2, g2, be2)):
        h = jnp.dot(h, w[...], preferred_element_type=jnp.float32) + b[...]
        m = jnp.mean(h, axis=0, keepdims=True)
        c = h - m
        v = jnp.mean(c * c, axis=0, keepdims=True)
        h = g[...] * c * lax.rsqrt(v + 1e-5) + be[...]
        h = jnp.maximum(h, 0.0)
    out_ref[...] = h


def _bottom_mlp(x, p):
    args = [x]
    for i in range(3):
        args += [p[f"bw{i}"], p[f"bb{i}"].reshape(1, -1),
                 p[f"bg{i}"].reshape(1, -1), p[f"bbeta{i}"].reshape(1, -1)]
    return pl.pallas_call(
        _bottom_body,
        out_shape=jax.ShapeDtypeStruct((B, E), jnp.float32),
    )(*args)


_BB = 128


def _top_body(h2_ref, embs_ref, w32t, m729t, tb0, tw1t, tb1, tw2t,
              tb2, tw3t, tb3, out_ref):
    h2t = h2_ref[...].T
    parts = [h2t] + [embs_ref[f, 0] for f in range(NF)]
    at = jnp.concatenate(parts, axis=0)
    a3 = at.reshape(NV, E, _BB)
    rows = []
    for n in range(NV):
        prod = a3 * a3[n][None]
        rows.append(jnp.sum(prod, axis=1))
    gt = jnp.concatenate(rows, axis=0)
    t = jnp.dot(w32t[...], h2t, preferred_element_type=jnp.float32)
    t = t + jnp.dot(m729t[...], gt, preferred_element_type=jnp.float32)
    t = jnp.maximum(t + tb0[...], 0.0)
    t = jnp.maximum(jnp.dot(tw1t[...], t, preferred_element_type=jnp.float32)
                    + tb1[...], 0.0)
    t = jnp.maximum(jnp.dot(tw2t[...], t, preferred_element_type=jnp.float32)
                    + tb2[...], 0.0)
    o = jnp.dot(tw3t[...], t, preferred_element_type=jnp.float32) + tb3[...]
    out_ref[...] = o[0, :]


def _top_mlp(h2, embs4, p):
    rows, cols = np.triu_indices(NV, k=1)
    tw0 = p["tw0"]
    w32t = tw0[:E, :].T
    m729 = jnp.zeros((NV * NV, tw0.shape[1]), jnp.float32)
    m729t = m729.at[rows * NV + cols, :].set(tw0[E:, :]).T
    weights = (w32t, m729t, p["tb0"].reshape(-1, 1),
               p["tw1"].T, p["tb1"].reshape(-1, 1),
               p["tw2"].T, p["tb2"].reshape(-1, 1),
               p["tw3"].T, p["tb3"].reshape(-1, 1))
    wspec = [pl.BlockSpec(w.shape, lambda i: (0, 0)) for w in weights]
    return pl.pallas_call(
        _top_body,
        grid=(B // _BB,),
        in_specs=[pl.BlockSpec((_BB, E), lambda i: (i, 0)),
                  pl.BlockSpec((NF, 1, E, 128), lambda i: (0, i, 0, 0))]
        + wspec,
        out_specs=pl.BlockSpec((_BB,), lambda i: (i,)),
        out_shape=jax.ShapeDtypeStruct((B,), jnp.float32),
    )(h2, embs4, *weights)


def kernel(dense_inputs, sparse_inputs, params):
    vt = sparse_inputs.astype(jnp.int32).T
    c = vt // VCHUNK
    r = vt - c * VCHUNK
    srt = (c * QS + r % QS).reshape(NF, 32, _CHUNK)
    offt = ((r // QS) * E).reshape(NF, 32, _CHUNK)
    pack = _repack(params["tables"])
    embs4 = _sc_gather(pack, srt, offt)
    h2 = _bottom_mlp(dense_inputs, params)
    return _top_mlp(h2, embs4, params)


# speedup vs baseline: 25.2873x; 1.0663x over previous
import functools

import numpy as np
import jax
import jax.numpy as jnp
from jax import lax
from jax.experimental import pallas as pl
from jax.experimental.pallas import tpu as pltpu
from jax.experimental.pallas import tpu_sc as plsc

B = 4096
NF = 26
VOC = 100001
E = 32
NV = NF + 1
NI = NV * (NV - 1) // 2

QS = 3200
VCHUNK = 4 * QS
NCH = 8
SRF = NCH * QS

_NC, _NS = 2, 16
_NW = _NC * _NS
_CHUNK = 128
_NCHUNK = B * NF // _CHUNK
_CPW = _NCHUNK // _NW


def _repack_body(b0, b1, b2, b3, out_ref):
    y = jnp.concatenate([b0[0], b1[0], b2[0], b3[0]], axis=0)
    out_ref[0] = y.T


def _repack(tables):
    tt = jnp.transpose(tables, (0, 2, 1))
    specs = [
        pl.BlockSpec((1, E, QS),
                     functools.partial(lambda q, f, c: (f, 0, 4 * c + q), q))
        for q in range(4)
    ]
    return pl.pallas_call(
        _repack_body,
        grid=(NF, NCH),
        in_specs=specs,
        out_specs=pl.BlockSpec((1, QS, 128), lambda f, c: (f, c, 0)),
        out_shape=jax.ShapeDtypeStruct((NF, SRF, 128), jnp.float32),
    )(tt, tt, tt, tt)


def _sc_gather(pack, srt, offt):
    mesh = plsc.VectorSubcoreMesh(core_axis_name="c", subcore_axis_name="s")

    @functools.partial(
        pl.kernel,
        mesh=mesh,
        out_type=jax.ShapeDtypeStruct((NF, 32, E, 128), jnp.float32),
        scratch_types=[
            pltpu.VMEM((2, _CHUNK), jnp.int32),
            pltpu.VMEM((2, _CHUNK), jnp.int32),
            pltpu.VMEM((2, _CHUNK, 128), jnp.float32),
            pltpu.VMEM((E, 128), jnp.float32),
            pltpu.SemaphoreType.DMA,
        ],
        compiler_params=pltpu.CompilerParams(use_tc_tiling_on_sc=False,
                                             needs_layout_passes=False),
    )
    def k(pack_hbm, srt_hbm, offt_hbm, out_hbm, idx_v, off_v, buf_v, ebuf_v,
          gsem):
        wid = lax.axis_index("s") * _NC + lax.axis_index("c")
        base = wid * _CPW

        def start_gather(j, p):
            c = base + j
            f = c // 32
            blk = c % 32
            pltpu.sync_copy(srt_hbm.at[f, blk], idx_v.at[p])
            pltpu.sync_copy(offt_hbm.at[f, blk], off_v.at[p])
            pltpu.async_copy(pack_hbm.at[f].at[idx_v.at[p]], buf_v.at[p],
                             gsem)

        start_gather(0, 0)

        def body(j, carry):
            p = j % 2
            c = base + j
            f = c // 32
            blk = c % 32

            @pl.when(j + 1 < _CPW)
            def _():
                start_gather(j + 1, (j + 1) % 2)

            pltpu.make_async_copy(pack_hbm.at[f, pl.ds(0, _CHUNK)],
                                  buf_v.at[p], gsem).wait()

            iota16 = jax.lax.iota(jnp.int32, 16)
            pvec = jnp.full((16,), p, jnp.int32)
            for g in range(8):
                rows = iota16 + g * 16
                off_g = off_v[p, pl.ds(g * 16, 16)]
                for d in range(E):
                    vals = plsc.load_gather(buf_v, [pvec, rows, off_g + d])
                    ebuf_v[d, pl.ds(g * 16, 16)] = vals
            pltpu.sync_copy(ebuf_v, out_hbm.at[f, blk])
            return carry

        lax.fori_loop(0, _CPW, body, 0)

    return k(pack, srt, offt)


def _bottom_body(x_ref, w0, b0, g0, be0, w1, b1, g1, be1, w2, b2, g2, be2,
                 out_ref):
    h = x_ref[...]
    for w, b, g, be in ((w0, b0, g0, be0), (w1, b1, g1, be1),
                        (w2, b2, g2, be2)):
        h = jnp.dot(h, w[...], preferred_element_type=jnp.float32) + b[...]
        m = jnp.mean(h, axis=0, keepdims=True)
        c = h - m
        v = jnp.mean(c * c, axis=0, keepdims=True)
        h = g[...] * c * lax.rsqrt(v + 1e-5) + be[...]
        h = jnp.maximum(h, 0.0)
    out_ref[...] = h


def _bottom_mlp(x, p):
    args = [x]
    for i in range(3):
        args += [p[f"bw{i}"], p[f"bb{i}"].reshape(1, -1),
                 p[f"bg{i}"].reshape(1, -1), p[f"bbeta{i}"].reshape(1, -1)]
    return pl.pallas_call(
        _bottom_body,
        out_shape=jax.ShapeDtypeStruct((B, E), jnp.float32),
    )(*args)


_BB = 128


def _top_body(h2_ref, embs_ref, w32t, m729t, tb0, tw1t, tb1, tw2t,
              tb2, tw3t, tb3, out_ref):
    h2t = h2_ref[...].T
    parts = [h2t] + [embs_ref[f, 0] for f in range(NF)]
    at = jnp.concatenate(parts, axis=0)
    a3 = at.reshape(NV, E, _BB)
    rows = []
    for n in range(NV):
        prod = a3 * a3[n][None]
        rows.append(jnp.sum(prod, axis=1))
    gt = jnp.concatenate(rows, axis=0)
    t = jnp.dot(w32t[...], h2t, preferred_element_type=jnp.float32)
    t = t + jnp.dot(m729t[...], gt, preferred_element_type=jnp.float32)
    t = jnp.maximum(t + tb0[...], 0.0)
    t = jnp.maximum(jnp.dot(tw1t[...], t, preferred_element_type=jnp.float32)
                    + tb1[...], 0.0)
    t = jnp.maximum(jnp.dot(tw2t[...], t, preferred_element_type=jnp.float32)
                    + tb2[...], 0.0)
    o = jnp.dot(tw3t[...], t, preferred_element_type=jnp.float32) + tb3[...]
    out_ref[...] = o[0, :]


def _top_mlp(h2, embs4, p):
    rows, cols = np.triu_indices(NV, k=1)
    tw0 = p["tw0"]
    w32t = tw0[:E, :].T
    m729 = jnp.zeros((NV * NV, tw0.shape[1]), jnp.float32)
    m729t = m729.at[rows * NV + cols, :].set(tw0[E:, :]).T
    weights = (w32t, m729t, p["tb0"].reshape(-1, 1),
               p["tw1"].T, p["tb1"].reshape(-1, 1),
               p["tw2"].T, p["tb2"].reshape(-1, 1),
               p["tw3"].T, p["tb3"].reshape(-1, 1))
    wspec = [pl.BlockSpec(w.shape, lambda i: (0, 0)) for w in weights]
    return pl.pallas_call(
        _top_body,
        grid=(B // _BB,),
        in_specs=[pl.BlockSpec((_BB, E), lambda i: (i, 0)),
                  pl.BlockSpec((NF, 1, E, 128), lambda i: (0, i, 0, 0))]
        + wspec,
        out_specs=pl.BlockSpec((_BB,), lambda i: (i,)),
        out_shape=jax.ShapeDtypeStruct((B,), jnp.float32),
    )(h2, embs4, *weights)


def kernel(dense_inputs, sparse_inputs, params):
    vt = sparse_inputs.astype(jnp.int32).T
    c = vt // VCHUNK
    r = vt - c * VCHUNK
    srt = (c * QS + r % QS).reshape(NF, 32, _CHUNK)
    offt = ((r // QS) * E).reshape(NF, 32, _CHUNK)
    pack = _repack(params["tables"])
    embs4 = _sc_gather(pack, srt, offt)
    h2 = _bottom_mlp(dense_inputs, params)
    return _top_mlp(h2, embs4, params)


# speedup vs baseline: 27.6667x; 1.0941x over previous
import functools

import numpy as np
import jax
import jax.numpy as jnp
from jax import lax
from jax.experimental import pallas as pl
from jax.experimental.pallas import tpu as pltpu
from jax.experimental.pallas import tpu_sc as plsc

B = 4096
NF = 26
VOC = 100001
E = 32
NV = NF + 1
NI = NV * (NV - 1) // 2

QS = 3200
VCHUNK = 4 * QS
NCH = 8
SRF = NCH * QS

_NC, _NS = 2, 16
_NW = _NC * _NS
_CHUNK = 128
_NCHUNK = B * NF // _CHUNK
_CPW = _NCHUNK // _NW


_NFH = NF // 2


def _repack_body(b0, b1, b2, b3, out_ref):
    y = jnp.concatenate([b0[0], b1[0], b2[0], b3[0]], axis=0)
    out_ref[0] = y.T


def _repack(tt, f0):
    specs = [
        pl.BlockSpec((1, E, QS),
                     functools.partial(
                         lambda q, f, c: (f0 + f, 0, 4 * c + q), q))
        for q in range(4)
    ]
    return pl.pallas_call(
        _repack_body,
        grid=(_NFH, NCH),
        in_specs=specs,
        out_specs=pl.BlockSpec((1, QS, 128), lambda f, c: (f, c, 0)),
        out_shape=jax.ShapeDtypeStruct((_NFH, SRF, 128), jnp.float32),
    )(tt, tt, tt, tt)


_CPWH = _NFH * 32 // _NW


def _sc_gather(pack, srt, offt):
    mesh = plsc.VectorSubcoreMesh(core_axis_name="c", subcore_axis_name="s")

    @functools.partial(
        pl.kernel,
        mesh=mesh,
        out_type=jax.ShapeDtypeStruct((_NFH, 32, E, 128), jnp.float32),
        scratch_types=[
            pltpu.VMEM((2, _CHUNK), jnp.int32),
            pltpu.VMEM((2, _CHUNK), jnp.int32),
            pltpu.VMEM((2, _CHUNK, 128), jnp.float32),
            pltpu.VMEM((E, 128), jnp.float32),
            pltpu.SemaphoreType.DMA,
        ],
        compiler_params=pltpu.CompilerParams(use_tc_tiling_on_sc=False,
                                             needs_layout_passes=False),
    )
    def k(pack_hbm, srt_hbm, offt_hbm, out_hbm, idx_v, off_v, buf_v, ebuf_v,
          gsem):
        wid = lax.axis_index("s") * _NC + lax.axis_index("c")
        base = wid * _CPWH

        def start_gather(j, p):
            c = base + j
            f = c // 32
            blk = c % 32
            pltpu.sync_copy(srt_hbm.at[f, blk], idx_v.at[p])
            pltpu.sync_copy(offt_hbm.at[f, blk], off_v.at[p])
            pltpu.async_copy(pack_hbm.at[f].at[idx_v.at[p]], buf_v.at[p],
                             gsem)

        start_gather(0, 0)

        def body(j, carry):
            p = j % 2
            c = base + j
            f = c // 32
            blk = c % 32

            @pl.when(j + 1 < _CPWH)
            def _():
                start_gather(j + 1, (j + 1) % 2)

            pltpu.make_async_copy(pack_hbm.at[f, pl.ds(0, _CHUNK)],
                                  buf_v.at[p], gsem).wait()

            iota16 = jax.lax.iota(jnp.int32, 16)
            pvec = jnp.full((16,), p, jnp.int32)
            for g in range(8):
                rows = iota16 + g * 16
                off_g = off_v[p, pl.ds(g * 16, 16)]
                for d in range(E):
                    vals = plsc.load_gather(buf_v, [pvec, rows, off_g + d])
                    ebuf_v[d, pl.ds(g * 16, 16)] = vals
            pltpu.sync_copy(ebuf_v, out_hbm.at[f, blk])
            return carry

        lax.fori_loop(0, _CPWH, body, 0)

    return k(pack, srt, offt)


def _bottom_body(x_ref, w0, b0, g0, be0, w1, b1, g1, be1, w2, b2, g2, be2,
                 out_ref):
    h = x_ref[...]
    for w, b, g, be in ((w0, b0, g0, be0), (w1, b1, g1, be1),
                        (w2, b2, g2, be2)):
        h = jnp.dot(h, w[...], preferred_element_type=jnp.float32) + b[...]
        m = jnp.mean(h, axis=0, keepdims=True)
        c = h - m
        v = jnp.mean(c * c, axis=0, keepdims=True)
        h = g[...] * c * lax.rsqrt(v + 1e-5) + be[...]
        h = jnp.maximum(h, 0.0)
    out_ref[...] = h


def _bottom_mlp(x, p):
    args = [x]
    for i in range(3):
        args += [p[f"bw{i}"], p[f"bb{i}"].reshape(1, -1),
                 p[f"bg{i}"].reshape(1, -1), p[f"bbeta{i}"].reshape(1, -1)]
    return pl.pallas_call(
        _bottom_body,
        out_shape=jax.ShapeDtypeStruct((B, E), jnp.float32),
    )(*args)


_BB = 128


def _top_body(h2_ref, ea_ref, eb_ref, w32t, m729t, tb0, tw1t, tb1, tw2t,
              tb2, tw3t, tb3, out_ref):
    h2t = h2_ref[...].T
    parts = ([h2t] + [ea_ref[f, 0] for f in range(_NFH)]
             + [eb_ref[f, 0] for f in range(_NFH)])
    at = jnp.concatenate(parts, axis=0)
    a3 = at.reshape(NV, E, _BB)
    rows = []
    for n in range(NV):
        prod = a3 * a3[n][None]
        rows.append(jnp.sum(prod, axis=1))
    gt = jnp.concatenate(rows, axis=0)
    t = jnp.dot(w32t[...], h2t, preferred_element_type=jnp.float32)
    t = t + jnp.dot(m729t[...], gt, preferred_element_type=jnp.float32)
    t = jnp.maximum(t + tb0[...], 0.0)
    t = jnp.maximum(jnp.dot(tw1t[...], t, preferred_element_type=jnp.float32)
                    + tb1[...], 0.0)
    t = jnp.maximum(jnp.dot(tw2t[...], t, preferred_element_type=jnp.float32)
                    + tb2[...], 0.0)
    o = jnp.dot(tw3t[...], t, preferred_element_type=jnp.float32) + tb3[...]
    out_ref[...] = o[0, :]


def _top_mlp(h2, embs_a, embs_b, p):
    rows, cols = np.triu_indices(NV, k=1)
    tw0 = p["tw0"]
    w32t = tw0[:E, :].T
    m729 = jnp.zeros((NV * NV, tw0.shape[1]), jnp.float32)
    m729t = m729.at[rows * NV + cols, :].set(tw0[E:, :]).T
    weights = (w32t, m729t, p["tb0"].reshape(-1, 1),
               p["tw1"].T, p["tb1"].reshape(-1, 1),
               p["tw2"].T, p["tb2"].reshape(-1, 1),
               p["tw3"].T, p["tb3"].reshape(-1, 1))
    wspec = [pl.BlockSpec(w.shape, lambda i: (0, 0)) for w in weights]
    return pl.pallas_call(
        _top_body,
        grid=(B // _BB,),
        in_specs=[pl.BlockSpec((_BB, E), lambda i: (i, 0)),
                  pl.BlockSpec((_NFH, 1, E, 128), lambda i: (0, i, 0, 0)),
                  pl.BlockSpec((_NFH, 1, E, 128), lambda i: (0, i, 0, 0))]
        + wspec,
        out_specs=pl.BlockSpec((_BB,), lambda i: (i,)),
        out_shape=jax.ShapeDtypeStruct((B,), jnp.float32),
    )(h2, embs_a, embs_b, *weights)


def kernel(dense_inputs, sparse_inputs, params):
    vt = sparse_inputs.astype(jnp.int32).T
    c = vt // VCHUNK
    r = vt - c * VCHUNK
    srt = (c * QS + r % QS).reshape(NF, 32, _CHUNK)
    offt = ((r // QS) * E).reshape(NF, 32, _CHUNK)
    tt = jnp.transpose(params["tables"], (0, 2, 1))
    pack_a = _repack(tt, 0)
    embs_a = _sc_gather(pack_a, srt[:_NFH], offt[:_NFH])
    pack_b = _repack(tt, _NFH)
    embs_b = _sc_gather(pack_b, srt[_NFH:], offt[_NFH:])
    h2 = _bottom_mlp(dense_inputs, params)
    return _top_mlp(h2, embs_a, embs_b, params)


# speedup vs baseline: 31.2692x; 1.1302x over previous
import functools

import numpy as np
import jax
import jax.numpy as jnp
from jax import lax
from jax.experimental import pallas as pl
from jax.experimental.pallas import tpu as pltpu
from jax.experimental.pallas import tpu_sc as plsc

B = 4096
NF = 26
VOC = 100001
E = 32
NV = NF + 1
NI = NV * (NV - 1) // 2

QS = 3200
VCHUNK = 4 * QS
NCH = 8
SRF = NCH * QS

_NC, _NS = 2, 16
_NW = _NC * _NS
_CHUNK = 128
_NCHUNK = B * NF // _CHUNK
_CPW = _NCHUNK // _NW


_NFH = NF // 2


def _repack_body(*refs):
    out_ref = refs[-1]
    for h in range(2):
        y = jnp.concatenate([refs[4 * h + q][0] for q in range(4)], axis=0)
        out_ref[0, h * QS:(h + 1) * QS, :] = y.T


def _repack(tt, f0):
    specs = [
        pl.BlockSpec((1, E, QS),
                     functools.partial(
                         lambda h, q, f, c: (f0 + f, 0, 8 * c + 4 * h + q),
                         h, q))
        for h in range(2) for q in range(4)
    ]
    return pl.pallas_call(
        _repack_body,
        grid=(_NFH, NCH // 2),
        in_specs=specs,
        out_specs=pl.BlockSpec((1, 2 * QS, 128), lambda f, c: (f, c, 0)),
        out_shape=jax.ShapeDtypeStruct((_NFH, SRF, 128), jnp.float32),
    )(*([tt] * 8))


_CPWH = _NFH * 32 // _NW


def _sc_gather(pack, srt, offt):
    mesh = plsc.VectorSubcoreMesh(core_axis_name="c", subcore_axis_name="s")

    @functools.partial(
        pl.kernel,
        mesh=mesh,
        out_type=jax.ShapeDtypeStruct((_NFH, 32, E, 128), jnp.float32),
        scratch_types=[
            pltpu.VMEM((2, _CHUNK), jnp.int32),
            pltpu.VMEM((2, _CHUNK), jnp.int32),
            pltpu.VMEM((2, _CHUNK, 128), jnp.float32),
            pltpu.VMEM((E, 128), jnp.float32),
            pltpu.SemaphoreType.DMA,
        ],
        compiler_params=pltpu.CompilerParams(use_tc_tiling_on_sc=False,
                                             needs_layout_passes=False),
    )
    def k(pack_hbm, srt_hbm, offt_hbm, out_hbm, idx_v, off_v, buf_v, ebuf_v,
          gsem):
        wid = lax.axis_index("s") * _NC + lax.axis_index("c")
        base = wid * _CPWH

        def start_gather(j, p):
            c = base + j
            f = c // 32
            blk = c % 32
            pltpu.sync_copy(srt_hbm.at[f, blk], idx_v.at[p])
            pltpu.sync_copy(offt_hbm.at[f, blk], off_v.at[p])
            pltpu.async_copy(pack_hbm.at[f].at[idx_v.at[p]], buf_v.at[p],
                             gsem)

        start_gather(0, 0)

        def body(j, carry):
            p = j % 2
            c = base + j
            f = c // 32
            blk = c % 32

            @pl.when(j + 1 < _CPWH)
            def _():
                start_gather(j + 1, (j + 1) % 2)

            pltpu.make_async_copy(pack_hbm.at[f, pl.ds(0, _CHUNK)],
                                  buf_v.at[p], gsem).wait()

            iota16 = jax.lax.iota(jnp.int32, 16)
            pvec = jnp.full((16,), p, jnp.int32)
            for g in range(8):
                rows = iota16 + g * 16
                off_g = off_v[p, pl.ds(g * 16, 16)]
                for d in range(E):
                    vals = plsc.load_gather(buf_v, [pvec, rows, off_g + d])
                    ebuf_v[d, pl.ds(g * 16, 16)] = vals
            pltpu.sync_copy(ebuf_v, out_hbm.at[f, blk])
            return carry

        lax.fori_loop(0, _CPWH, body, 0)

    return k(pack, srt, offt)


def _bottom_body(x_ref, w0, b0, g0, be0, w1, b1, g1, be1, w2, b2, g2, be2,
                 out_ref):
    h = x_ref[...]
    for w, b, g, be in ((w0, b0, g0, be0), (w1, b1, g1, be1),
                        (w2, b2, g2, be2)):
        h = jnp.dot(h, w[...], preferred_element_type=jnp.float32) + b[...]
        m = jnp.mean(h, axis=0, keepdims=True)
        c = h - m
        v = jnp.mean(c * c, axis=0, keepdims=True)
        h = g[...] * c * lax.rsqrt(v + 1e-5) + be[...]
        h = jnp.maximum(h, 0.0)
    out_ref[...] = h


def _bottom_mlp(x, p):
    args = [x]
    for i in range(3):
        args += [p[f"bw{i}"], p[f"bb{i}"].reshape(1, -1),
                 p[f"bg{i}"].reshape(1, -1), p[f"bbeta{i}"].reshape(1, -1)]
    return pl.pallas_call(
        _bottom_body,
        out_shape=jax.ShapeDtypeStruct((B, E), jnp.float32),
    )(*args)


_BB = 128


def _top_body(h2_ref, ea_ref, eb_ref, w32t, m729t, tb0, tw1t, tb1, tw2t,
              tb2, tw3t, tb3, out_ref):
    h2t = h2_ref[...].T
    parts = ([h2t] + [ea_ref[f, 0] for f in range(_NFH)]
             + [eb_ref[f, 0] for f in range(_NFH)])
    at = jnp.concatenate(parts, axis=0)
    a3 = at.reshape(NV, E, _BB)
    rows = []
    for n in range(NV):
        prod = a3 * a3[n][None]
        rows.append(jnp.sum(prod, axis=1))
    gt = jnp.concatenate(rows, axis=0)
    t = jnp.dot(w32t[...], h2t, preferred_element_type=jnp.float32)
    t = t + jnp.dot(m729t[...], gt, preferred_element_type=jnp.float32)
    t = jnp.maximum(t + tb0[...], 0.0)
    t = jnp.maximum(jnp.dot(tw1t[...], t, preferred_element_type=jnp.float32)
                    + tb1[...], 0.0)
    t = jnp.maximum(jnp.dot(tw2t[...], t, preferred_element_type=jnp.float32)
                    + tb2[...], 0.0)
    o = jnp.dot(tw3t[...], t, preferred_element_type=jnp.float32) + tb3[...]
    out_ref[...] = o[0, :]


def _top_mlp(h2, embs_a, embs_b, p):
    rows, cols = np.triu_indices(NV, k=1)
    tw0 = p["tw0"]
    w32t = tw0[:E, :].T
    m729 = jnp.zeros((NV * NV, tw0.shape[1]), jnp.float32)
    m729t = m729.at[rows * NV + cols, :].set(tw0[E:, :]).T
    weights = (w32t, m729t, p["tb0"].reshape(-1, 1),
               p["tw1"].T, p["tb1"].reshape(-1, 1),
               p["tw2"].T, p["tb2"].reshape(-1, 1),
               p["tw3"].T, p["tb3"].reshape(-1, 1))
    wspec = [pl.BlockSpec(w.shape, lambda i: (0, 0)) for w in weights]
    return pl.pallas_call(
        _top_body,
        grid=(B // _BB,),
        in_specs=[pl.BlockSpec((_BB, E), lambda i: (i, 0)),
                  pl.BlockSpec((_NFH, 1, E, 128), lambda i: (0, i, 0, 0)),
                  pl.BlockSpec((_NFH, 1, E, 128), lambda i: (0, i, 0, 0))]
        + wspec,
        out_specs=pl.BlockSpec((_BB,), lambda i: (i,)),
        out_shape=jax.ShapeDtypeStruct((B,), jnp.float32),
    )(h2, embs_a, embs_b, *weights)


def kernel(dense_inputs, sparse_inputs, params):
    vt = sparse_inputs.astype(jnp.int32).T
    c = vt // VCHUNK
    r = vt - c * VCHUNK
    srt = (c * QS + r % QS).reshape(NF, 32, _CHUNK)
    offt = ((r // QS) * E).reshape(NF, 32, _CHUNK)
    tt = jnp.transpose(params["tables"], (0, 2, 1))
    pack_a = _repack(tt, 0)
    embs_a = _sc_gather(pack_a, srt[:_NFH], offt[:_NFH])
    pack_b = _repack(tt, _NFH)
    embs_b = _sc_gather(pack_b, srt[_NFH:], offt[_NFH:])
    h2 = _bottom_mlp(dense_inputs, params)
    return _top_mlp(h2, embs_a, embs_b, params)


# speedup vs baseline: 32.6484x; 1.0441x over previous
import functools

import numpy as np
import jax
import jax.numpy as jnp
from jax import lax
from jax.experimental import pallas as pl
from jax.experimental.pallas import tpu as pltpu
from jax.experimental.pallas import tpu_sc as plsc

B = 4096
NF = 26
VOC = 100001
E = 32
NV = NF + 1
NI = NV * (NV - 1) // 2

QS = 3200
VCHUNK = 4 * QS
NCH = 8
SRF = NCH * QS

_NC, _NS = 2, 16
_NW = _NC * _NS
_CHUNK = 128
_NCHUNK = B * NF // _CHUNK
_CPW = _NCHUNK // _NW


_NFH = NF // 2


def _repack_body(*refs):
    out_ref = refs[-1]
    for h in range(2):
        y = jnp.concatenate([refs[4 * h + q][0] for q in range(4)], axis=0)
        out_ref[0, h * QS:(h + 1) * QS, :] = y.T


def _repack(tt, f0, nf):
    specs = [
        pl.BlockSpec((1, E, QS),
                     functools.partial(
                         lambda h, q, f, c: (f0 + f, 0, 8 * c + 4 * h + q),
                         h, q))
        for h in range(2) for q in range(4)
    ]
    return pl.pallas_call(
        _repack_body,
        grid=(nf, NCH // 2),
        in_specs=specs,
        out_specs=pl.BlockSpec((1, 2 * QS, 128), lambda f, c: (f, c, 0)),
        out_shape=jax.ShapeDtypeStruct((nf, SRF, 128), jnp.float32),
    )(*([tt] * 8))


def _sc_gather(pack, srt, offt, nf):
    mesh = plsc.VectorSubcoreMesh(core_axis_name="c", subcore_axis_name="s")

    @functools.partial(
        pl.kernel,
        mesh=mesh,
        out_type=jax.ShapeDtypeStruct((nf, 32, E, 128), jnp.float32),
        scratch_types=[
            pltpu.VMEM((2, _CHUNK), jnp.int32),
            pltpu.VMEM((2, _CHUNK), jnp.int32),
            pltpu.VMEM((2, _CHUNK, 128), jnp.float32),
            pltpu.VMEM((E, 128), jnp.float32),
            pltpu.SemaphoreType.DMA,
        ],
        compiler_params=pltpu.CompilerParams(use_tc_tiling_on_sc=False,
                                             needs_layout_passes=False),
    )
    def k(pack_hbm, srt_hbm, offt_hbm, out_hbm, idx_v, off_v, buf_v, ebuf_v,
          gsem):
        wid = lax.axis_index("s") * _NC + lax.axis_index("c")
        base = wid * nf

        def start_gather(j, p):
            c = base + j
            f = c // 32
            blk = c % 32
            pltpu.sync_copy(srt_hbm.at[f, blk], idx_v.at[p])
            pltpu.sync_copy(offt_hbm.at[f, blk], off_v.at[p])
            pltpu.async_copy(pack_hbm.at[f].at[idx_v.at[p]], buf_v.at[p],
                             gsem)

        start_gather(0, 0)

        def body(j, carry):
            p = j % 2
            c = base + j
            f = c // 32
            blk = c % 32

            @pl.when(j + 1 < nf)
            def _():
                start_gather(j + 1, (j + 1) % 2)

            pltpu.make_async_copy(pack_hbm.at[f, pl.ds(0, _CHUNK)],
                                  buf_v.at[p], gsem).wait()

            iota16 = jax.lax.iota(jnp.int32, 16)
            pvec = jnp.full((16,), p, jnp.int32)
            for g in range(8):
                rows = iota16 + g * 16
                off_g = off_v[p, pl.ds(g * 16, 16)]
                for d in range(E):
                    vals = plsc.load_gather(buf_v, [pvec, rows, off_g + d])
                    ebuf_v[d, pl.ds(g * 16, 16)] = vals
            pltpu.sync_copy(ebuf_v, out_hbm.at[f, blk])
            return carry

        lax.fori_loop(0, nf, body, 0)

    return k(pack, srt, offt)


def _bottom_body(x_ref, w0, b0, g0, be0, w1, b1, g1, be1, w2, b2, g2, be2,
                 out_ref):
    h = x_ref[...]
    for w, b, g, be in ((w0, b0, g0, be0), (w1, b1, g1, be1),
                        (w2, b2, g2, be2)):
        h = jnp.dot(h, w[...], preferred_element_type=jnp.float32) + b[...]
        m = jnp.mean(h, axis=0, keepdims=True)
        c = h - m
        v = jnp.mean(c * c, axis=0, keepdims=True)
        h = g[...] * c * lax.rsqrt(v + 1e-5) + be[...]
        h = jnp.maximum(h, 0.0)
    out_ref[...] = h


def _bottom_mlp(x, p):
    args = [x]
    for i in range(3):
        args += [p[f"bw{i}"], p[f"bb{i}"].reshape(1, -1),
                 p[f"bg{i}"].reshape(1, -1), p[f"bbeta{i}"].reshape(1, -1)]
    return pl.pallas_call(
        _bottom_body,
        out_shape=jax.ShapeDtypeStruct((B, E), jnp.float32),
    )(*args)


_BB = 128


_SLICES = (7, 7, 6, 6)


def _top_body(h2_ref, ea_ref, eb_ref, ec_ref, ed_ref, w32t, m729t, tb0,
              tw1t, tb1, tw2t, tb2, tw3t, tb3, out_ref):
    h2t = h2_ref[...].T
    parts = [h2t]
    for eref, nf in zip((ea_ref, eb_ref, ec_ref, ed_ref), _SLICES):
        parts += [eref[f, 0] for f in range(nf)]
    at = jnp.concatenate(parts, axis=0)
    a3 = at.reshape(NV, E, _BB)
    rows = []
    for n in range(NV):
        prod = a3 * a3[n][None]
        rows.append(jnp.sum(prod, axis=1))
    gt = jnp.concatenate(rows, axis=0)
    t = jnp.dot(w32t[...], h2t, preferred_element_type=jnp.float32)
    t = t + jnp.dot(m729t[...], gt, preferred_element_type=jnp.float32)
    t = jnp.maximum(t + tb0[...], 0.0)
    t = jnp.maximum(jnp.dot(tw1t[...], t, preferred_element_type=jnp.float32)
                    + tb1[...], 0.0)
    t = jnp.maximum(jnp.dot(tw2t[...], t, preferred_element_type=jnp.float32)
                    + tb2[...], 0.0)
    o = jnp.dot(tw3t[...], t, preferred_element_type=jnp.float32) + tb3[...]
    out_ref[...] = o[0, :]


def _top_mlp(h2, embs_slices, p):
    rows, cols = np.triu_indices(NV, k=1)
    tw0 = p["tw0"]
    w32t = tw0[:E, :].T
    m729 = jnp.zeros((NV * NV, tw0.shape[1]), jnp.float32)
    m729t = m729.at[rows * NV + cols, :].set(tw0[E:, :]).T
    weights = (w32t, m729t, p["tb0"].reshape(-1, 1),
               p["tw1"].T, p["tb1"].reshape(-1, 1),
               p["tw2"].T, p["tb2"].reshape(-1, 1),
               p["tw3"].T, p["tb3"].reshape(-1, 1))
    wspec = [pl.BlockSpec(w.shape, lambda i: (0, 0)) for w in weights]
    return pl.pallas_call(
        _top_body,
        grid=(B // _BB,),
        in_specs=[pl.BlockSpec((_BB, E), lambda i: (i, 0))]
        + [pl.BlockSpec((nf, 1, E, 128), lambda i: (0, i, 0, 0))
           for nf in _SLICES]
        + wspec,
        out_specs=pl.BlockSpec((_BB,), lambda i: (i,)),
        out_shape=jax.ShapeDtypeStruct((B,), jnp.float32),
    )(h2, *embs_slices, *weights)


def kernel(dense_inputs, sparse_inputs, params):
    vt = sparse_inputs.astype(jnp.int32).T
    c = vt // VCHUNK
    r = vt - c * VCHUNK
    srt = (c * QS + r % QS).reshape(NF, 32, _CHUNK)
    offt = ((r // QS) * E).reshape(NF, 32, _CHUNK)
    tt = jnp.transpose(params["tables"], (0, 2, 1))
    embs_slices = []
    f0 = 0
    for nf in _SLICES:
        pack = _repack(tt, f0, nf)
        embs_slices.append(
            _sc_gather(pack, srt[f0:f0 + nf], offt[f0:f0 + nf], nf))
        f0 += nf
    h2 = _bottom_mlp(dense_inputs, params)
    return _top_mlp(h2, embs_slices, params)


# speedup vs baseline: 32.6775x; 1.0009x over previous
import functools

import numpy as np
import jax
import jax.numpy as jnp
from jax import lax
from jax.experimental import pallas as pl
from jax.experimental.pallas import tpu as pltpu
from jax.experimental.pallas import tpu_sc as plsc

B = 4096
NF = 26
VOC = 100001
E = 32
NV = NF + 1
NI = NV * (NV - 1) // 2

QS = 3200
VCHUNK = 4 * QS
NCH = 8
SRF = NCH * QS

_NC, _NS = 2, 16
_NW = _NC * _NS
_CHUNK = 128


def _repack_body(*refs):
    out_ref = refs[-1]
    for h in range(2):
        y = jnp.concatenate([refs[4 * h + q][0] for q in range(4)], axis=0)
        out_ref[0, h * QS:(h + 1) * QS, :] = y.T


def _repack(tt, f0, nf):
    specs = [
        pl.BlockSpec((1, E, QS),
                     functools.partial(
                         lambda h, q, f, c: (f0 + f, 0, 8 * c + 4 * h + q),
                         h, q))
        for h in range(2) for q in range(4)
    ]
    return pl.pallas_call(
        _repack_body,
        grid=(nf, NCH // 2),
        in_specs=specs,
        out_specs=pl.BlockSpec((1, 2 * QS, 128), lambda f, c: (f, c, 0)),
        out_shape=jax.ShapeDtypeStruct((nf, SRF, 128), jnp.float32),
    )(*([tt] * 8))


def _sc_gather(pack, srt, offt, nf):
    mesh = plsc.VectorSubcoreMesh(core_axis_name="c", subcore_axis_name="s")

    @functools.partial(
        pl.kernel,
        mesh=mesh,
        out_type=jax.ShapeDtypeStruct((nf, 32, E, 128), jnp.float32),
        scratch_types=[
            pltpu.VMEM((2, _CHUNK), jnp.int32),
            pltpu.VMEM((2, _CHUNK), jnp.int32),
            pltpu.VMEM((2, _CHUNK, 128), jnp.float32),
            pltpu.VMEM((E, 128), jnp.float32),
            pltpu.SemaphoreType.DMA,
        ],
        compiler_params=pltpu.CompilerParams(use_tc_tiling_on_sc=False,
                                             needs_layout_passes=False),
    )
    def k(pack_hbm, srt_hbm, offt_hbm, out_hbm, idx_v, off_v, buf_v, ebuf_v,
          gsem):
        wid = lax.axis_index("s") * _NC + lax.axis_index("c")
        base = wid * nf

        def start_gather(j, p):
            c = base + j
            f = c // 32
            blk = c % 32
            pltpu.sync_copy(srt_hbm.at[f, blk], idx_v.at[p])
            pltpu.sync_copy(offt_hbm.at[f, blk], off_v.at[p])
            pltpu.async_copy(pack_hbm.at[f].at[idx_v.at[p]], buf_v.at[p],
                             gsem)

        start_gather(0, 0)

        def body(j, carry):
            p = j % 2
            c = base + j
            f = c // 32
            blk = c % 32

            @pl.when(j + 1 < nf)
            def _():
                start_gather(j + 1, (j + 1) % 2)

            pltpu.make_async_copy(pack_hbm.at[f, pl.ds(0, _CHUNK)],
                                  buf_v.at[p], gsem).wait()

            iota16 = jax.lax.iota(jnp.int32, 16)
            pvec = jnp.full((16,), p, jnp.int32)
            for g in range(8):
                rows = iota16 + g * 16
                off_g = off_v[p, pl.ds(g * 16, 16)]
                for d in range(E):
                    vals = plsc.load_gather(buf_v, [pvec, rows, off_g + d])
                    ebuf_v[d, pl.ds(g * 16, 16)] = vals
            pltpu.sync_copy(ebuf_v, out_hbm.at[f, blk])
            return carry

        lax.fori_loop(0, nf, body, 0)

    return k(pack, srt, offt)


def _bottom_body(x_ref, w0, b0, g0, be0, w1, b1, g1, be1, w2, b2, g2, be2,
                 out_ref):
    h = x_ref[...]
    for w, b, g, be in ((w0, b0, g0, be0), (w1, b1, g1, be1),
                        (w2, b2, g2, be2)):
        h = jnp.dot(h, w[...], preferred_element_type=jnp.float32) + b[...]
        m = jnp.mean(h, axis=0, keepdims=True)
        c = h - m
        v = jnp.mean(c * c, axis=0, keepdims=True)
        h = g[...] * c * lax.rsqrt(v + 1e-5) + be[...]
        h = jnp.maximum(h, 0.0)
    out_ref[...] = h


def _bottom_mlp(x, p):
    args = [x]
    for i in range(3):
        args += [p[f"bw{i}"], p[f"bb{i}"].reshape(1, -1),
                 p[f"bg{i}"].reshape(1, -1), p[f"bbeta{i}"].reshape(1, -1)]
    return pl.pallas_call(
        _bottom_body,
        out_shape=jax.ShapeDtypeStruct((B, E), jnp.float32),
    )(*args)


_BB = 128


_SLICES = (7, 7, 6, 6)


def _top_body(h2_ref, ea_ref, eb_ref, ec_ref, ed_ref, w32t, m729t, tb0,
              tw1t, tb1, tw2t, tb2, tw3t, tb3, out_ref):
    h2t = h2_ref[...].T
    parts = [h2t]
    for eref, nf in zip((ea_ref, eb_ref, ec_ref, ed_ref), _SLICES):
        parts += [eref[f, 0] for f in range(nf)]
    at = jnp.concatenate(parts, axis=0)
    a3 = at.reshape(NV, E, _BB)
    rows = []
    for n in range(NV):
        prod = a3 * a3[n][None]
        rows.append(jnp.sum(prod, axis=1))
    gt = jnp.concatenate(rows, axis=0)
    t = jnp.dot(w32t[...], h2t, preferred_element_type=jnp.float32)
    t = t + jnp.dot(m729t[...], gt, preferred_element_type=jnp.float32)
    t = jnp.maximum(t + tb0[...], 0.0)
    t = jnp.maximum(jnp.dot(tw1t[...], t, preferred_element_type=jnp.float32)
                    + tb1[...], 0.0)
    t = jnp.maximum(jnp.dot(tw2t[...], t, preferred_element_type=jnp.float32)
                    + tb2[...], 0.0)
    o = jnp.dot(tw3t[...], t, preferred_element_type=jnp.float32) + tb3[...]
    out_ref[...] = o[0, :]


def _top_mlp(h2, embs_slices, p):
    rows, cols = np.triu_indices(NV, k=1)
    tw0 = p["tw0"]
    w32t = tw0[:E, :].T
    m729 = jnp.zeros((NV * NV, tw0.shape[1]), jnp.float32)
    m729t = m729.at[rows * NV + cols, :].set(tw0[E:, :]).T
    weights = (w32t, m729t, p["tb0"].reshape(-1, 1),
               p["tw1"].T, p["tb1"].reshape(-1, 1),
               p["tw2"].T, p["tb2"].reshape(-1, 1),
               p["tw3"].T, p["tb3"].reshape(-1, 1))
    wspec = [pl.BlockSpec(w.shape, lambda i: (0, 0)) for w in weights]
    return pl.pallas_call(
        _top_body,
        grid=(B // _BB,),
        in_specs=[pl.BlockSpec((_BB, E), lambda i: (i, 0))]
        + [pl.BlockSpec((nf, 1, E, 128), lambda i: (0, i, 0, 0))
           for nf in _SLICES]
        + wspec,
        out_specs=pl.BlockSpec((_BB,), lambda i: (i,)),
        out_shape=jax.ShapeDtypeStruct((B,), jnp.float32),
    )(h2, *embs_slices, *weights)


def kernel(dense_inputs, sparse_inputs, params):
    vt = sparse_inputs.astype(jnp.int32).T
    c = vt // VCHUNK
    r = vt - c * VCHUNK
    srt = (c * QS + r % QS).reshape(NF, 32, _CHUNK)
    offt = ((r // QS) * E).reshape(NF, 32, _CHUNK)
    tt = jnp.transpose(params["tables"], (0, 2, 1))
    embs_slices = []
    f0 = 0
    for nf in _SLICES:
        pack = _repack(tt, f0, nf)
        embs_slices.append(
            _sc_gather(pack, srt[f0:f0 + nf], offt[f0:f0 + nf], nf))
        f0 += nf
    h2 = _bottom_mlp(dense_inputs, params)
    return _top_mlp(h2, embs_slices, params)
